# baseline (speedup 1.0000x reference)
import functools

import jax
import jax.numpy as jnp
from jax import lax
from jax.experimental import pallas as pl
from jax.experimental.pallas import tpu as pltpu
from jax.experimental.pallas import tpu_sc as plsc

B, L, HIDDEN = 1024, 200, 128
VOCAB = 100000
NB0 = 50000
BL = B * L


_BLK = 2000
_NBLK0 = NB0 // _BLK


def _big_table_body(emb0_ref, emb1_ref, factor1_ref, out_ref):
    i = pl.program_id(0)

    @pl.when(i < _NBLK0)
    def _copy():
        out_ref[...] = emb0_ref[...].astype(jnp.bfloat16)

    @pl.when(i >= _NBLK0)
    def _proj():
        out_ref[...] = jnp.dot(emb1_ref[...], factor1_ref[...],
                               preferred_element_type=jnp.float32
                               ).astype(jnp.bfloat16)


def _build_big_table(emb0, emb1, factor1):
    return pl.pallas_call(
        _big_table_body,
        grid=(2 * _NBLK0,),
        in_specs=[
            pl.BlockSpec((_BLK, HIDDEN), lambda i: (jnp.minimum(i, _NBLK0 - 1), 0)),
            pl.BlockSpec((_BLK, 32), lambda i: (jnp.maximum(i - _NBLK0, 0), 0)),
            pl.BlockSpec((32, HIDDEN), lambda i: (0, 0)),
        ],
        out_specs=pl.BlockSpec((_BLK, HIDDEN), lambda i: (i, 0)),
        out_shape=jax.ShapeDtypeStruct((VOCAB, HIDDEN), jnp.bfloat16),
    )(emb0, emb1, factor1)


def _small_table_body(count_ref, value_ref, io_ref, posw_ref, s_out, pos_out):
    s = (count_ref[...][:, None, None, :]
         + value_ref[...][None, :, None, :]
         + io_ref[...][None, None, :, :])
    s_out[...] = s.reshape(675, HIDDEN).astype(jnp.bfloat16)
    pos_out[...] = posw_ref[...].astype(jnp.bfloat16)


def _build_small_tables(count_w, value_w, io_w, pos_w):
    return pl.pallas_call(
        _small_table_body,
        out_shape=(jax.ShapeDtypeStruct((675, HIDDEN), jnp.bfloat16),
                   jax.ShapeDtypeStruct((L, HIDDEN), jnp.bfloat16)),
    )(count_w, value_w, io_w, pos_w)


_NW = 32
_TPW = BL // _NW
_CH = 128
_NCH = _TPW // _CH
_NCC = _NCH // 2

def _bf16_halves(w):
    v = plsc.bitcast(w, jnp.bfloat16)
    return plsc.unpack(v, format=plsc.PackFormat.INTERLEAVED)


def _sc_body(big_hbm, s_hbm, posw_hbm, ids_hbm, cnt_hbm, val_hbm, io_hbm,
             pos_hbm, out_hbm,
             ids_loc, vci_loc, val_loc, io_loc, pos_loc,
             brow0, brow1, srow0, srow1, prow0, prow1, obuf0, obuf1,
             g00, g10, g20, g01, g11, g21, o0, o1):
    wid = lax.axis_index("s") * 2 + lax.axis_index("c")
    wbase = wid * _TPW

    pltpu.sync_copy(ids_hbm.at[wid], ids_loc)
    pltpu.sync_copy(cnt_hbm.at[wid], vci_loc)
    pltpu.sync_copy(val_hbm.at[wid], val_loc)
    pltpu.sync_copy(io_hbm.at[wid], io_loc)
    pltpu.sync_copy(pos_hbm.at[wid], pos_loc)

    def vci_row(r, carry):
        for k in range(_CH // 16):
            sl = pl.ds(k * 16, 16)
            vci_loc[r, sl] = (vci_loc[r, sl] * 45 + val_loc[r, sl] * 3
                              + io_loc[r, sl])
        return carry

    lax.fori_loop(0, _NCH, vci_row, 0)

    sets = ((brow0, srow0, prow0, obuf0, g00, g10, g20, o0),
            (brow1, srow1, prow1, obuf1, g01, g11, g21, o1))

    def fire_gathers(c, bset):
        brow, srow, prow = bset[0], bset[1], bset[2]
        pltpu.async_copy(big_hbm.at[ids_loc.at[c]], brow, bset[4])
        pltpu.async_copy(s_hbm.at[vci_loc.at[c]], srow, bset[5])
        pltpu.async_copy(posw_hbm.at[pos_loc.at[c]], prow, bset[6])

    def wait_gathers(c, bset):
        pltpu.make_async_copy(big_hbm.at[ids_loc.at[c]], bset[0], bset[4]).wait()
        pltpu.make_async_copy(s_hbm.at[vci_loc.at[c]], bset[1], bset[5]).wait()
        pltpu.make_async_copy(posw_hbm.at[pos_loc.at[c]], bset[2], bset[6]).wait()

    fire_gathers(0, sets[0])
    fire_gathers(1, sets[1])

    lane = lax.iota(jnp.int32, 16)
    evens = [lane * 2 + 32 * j for j in range(4)]
    odds = [lane * 2 + 32 * j + 1 for j in range(4)]

    def loop_body(cc, carry):
        for b in (0, 1):
            brow, srow, prow, obuf, _, _, _, osem = sets[b]
            c = 2 * cc + b
            base = wbase + c * _CH
            out_slice = out_hbm.at[pl.ds(base, _CH)]
            wait_gathers(c, sets[b])

            @pl.when(cc > 0)
            def _drain():
                pltpu.make_async_copy(obuf, out_slice, osem).wait()

            def row_body(t, carry2):
                trow = jnp.broadcast_to(t, (16,)).astype(jnp.int32)
                for j in range(4):
                    sl = pl.ds(j * 16, 16)
                    blo, bhi = _bf16_halves(brow[t, sl])
                    slo, shi = _bf16_halves(srow[t, sl])
                    plo, phi = _bf16_halves(prow[t, sl])
                    plsc.store_scatter(obuf, [trow, evens[j]], blo + slo + plo)
                    plsc.store_scatter(obuf, [trow, odds[j]], bhi + shi + phi)
                return carry2

            lax.fori_loop(0, _CH, row_body, 0)
            pltpu.async_copy(obuf, out_slice, osem)

            @pl.when(cc < _NCC - 1)
            def _prefetch():
                fire_gathers(c + 2, sets[b])
        return carry

    lax.fori_loop(0, _NCC, loop_body, 0)

    for b in (0, 1):
        obuf, osem = sets[b][3], sets[b][7]
        pltpu.make_async_copy(obuf, out_hbm.at[pl.ds(wbase, _CH)], osem).wait()


_sc_gather = functools.partial(
    pl.kernel,
    out_type=jax.ShapeDtypeStruct((BL, HIDDEN), jnp.float32),
    mesh=plsc.VectorSubcoreMesh(core_axis_name="c", subcore_axis_name="s"),
    compiler_params=pltpu.CompilerParams(needs_layout_passes=False,
                                         use_tc_tiling_on_sc=False),
    scratch_types=[
        pltpu.VMEM((_NCH, _CH), jnp.int32),
        pltpu.VMEM((_NCH, _CH), jnp.int32),
        pltpu.VMEM((_NCH, _CH), jnp.int32),
        pltpu.VMEM((_NCH, _CH), jnp.int32),
        pltpu.VMEM((_NCH, _CH), jnp.int32),
        pltpu.VMEM((_CH, HIDDEN // 2), jnp.int32),
        pltpu.VMEM((_CH, HIDDEN // 2), jnp.int32),
        pltpu.VMEM((_CH, HIDDEN // 2), jnp.int32),
        pltpu.VMEM((_CH, HIDDEN // 2), jnp.int32),
        pltpu.VMEM((_CH, HIDDEN // 2), jnp.int32),
        pltpu.VMEM((_CH, HIDDEN // 2), jnp.int32),
        pltpu.VMEM((_CH, HIDDEN), jnp.float32),
        pltpu.VMEM((_CH, HIDDEN), jnp.float32),
        pltpu.SemaphoreType.DMA,
        pltpu.SemaphoreType.DMA,
        pltpu.SemaphoreType.DMA,
        pltpu.SemaphoreType.DMA,
        pltpu.SemaphoreType.DMA,
        pltpu.SemaphoreType.DMA,
        pltpu.SemaphoreType.DMA,
        pltpu.SemaphoreType.DMA,
    ],
)(_sc_body)


def kernel(input_ids, counts, values, io_flags, positions,
           emb0, emb1, factor1, value_w, count_w, pos_w, io_w):
    big = _build_big_table(emb0, emb1, factor1)
    s_tab, pos_tab = _build_small_tables(count_w, value_w, io_w, pos_w)
    shp = (_NW, _NCH, _CH)
    big_i = lax.bitcast_convert_type(big.reshape(VOCAB, 64, 2), jnp.int32)
    s_i = lax.bitcast_convert_type(s_tab.reshape(675, 64, 2), jnp.int32)
    pos_i = lax.bitcast_convert_type(pos_tab.reshape(L, 64, 2), jnp.int32)
    out = _sc_gather(big_i, s_i, pos_i,
                     input_ids.reshape(shp), counts.reshape(shp),
                     values.reshape(shp), io_flags.reshape(shp),
                     positions.reshape(shp))
    return out.reshape(B, L, HIDDEN)


# speedup vs baseline: 2.6941x; 2.6941x over previous
import functools

import jax
import jax.numpy as jnp
from jax import lax
from jax.experimental import pallas as pl
from jax.experimental.pallas import tpu as pltpu
from jax.experimental.pallas import tpu_sc as plsc

B, L, HIDDEN = 1024, 200, 128
VOCAB = 100000
NB0 = 50000
BL = B * L


_BLK = 5000
_NBLK0 = NB0 // _BLK


def _pack_bf16_words(x):
    lo = lax.bitcast_convert_type(x[:, :HIDDEN // 2], jnp.int32)
    hi = lax.bitcast_convert_type(x[:, HIDDEN // 2:], jnp.int32)
    return lax.shift_right_logical(lo, 16) | (hi & -65536)


def _table_body(emb0_ref, emb1_ref, factor1_ref, count_ref, value_ref,
                io_ref, posw_ref, big_out, s_out, pos_out):
    i = pl.program_id(0)

    @pl.when(i < _NBLK0)
    def _copy():
        big_out[...] = emb0_ref[...]

    @pl.when(i >= _NBLK0)
    def _proj():
        big_out[...] = jnp.dot(emb1_ref[...], factor1_ref[...],
                               preferred_element_type=jnp.float32)

    @pl.when(i == 0)
    def _small():
        s = (count_ref[...][:, None, None, :]
             + value_ref[...][None, :, None, :]
             + io_ref[...][None, None, :, :])
        s_out[...] = _pack_bf16_words(s.reshape(675, HIDDEN))
        pos_out[...] = _pack_bf16_words(posw_ref[...])


def _build_tables(emb0, emb1, factor1, count_w, value_w, io_w, pos_w):
    zero = lambda i: (0, 0)
    return pl.pallas_call(
        _table_body,
        grid=(2 * _NBLK0,),
        in_specs=[
            pl.BlockSpec((_BLK, HIDDEN), lambda i: (jnp.minimum(i, _NBLK0 - 1), 0)),
            pl.BlockSpec((_BLK, 32), lambda i: (jnp.maximum(i - _NBLK0, 0), 0)),
            pl.BlockSpec((32, HIDDEN), zero),
            pl.BlockSpec((15, HIDDEN), zero),
            pl.BlockSpec((15, HIDDEN), zero),
            pl.BlockSpec((3, HIDDEN), zero),
            pl.BlockSpec((L, HIDDEN), zero),
        ],
        out_specs=(pl.BlockSpec((_BLK, HIDDEN), lambda i: (i, 0)),
                   pl.BlockSpec((675, HIDDEN // 2), zero),
                   pl.BlockSpec((L, HIDDEN // 2), zero)),
        out_shape=(jax.ShapeDtypeStruct((VOCAB, HIDDEN), jnp.float32),
                   jax.ShapeDtypeStruct((675, HIDDEN // 2), jnp.int32),
                   jax.ShapeDtypeStruct((L, HIDDEN // 2), jnp.int32)),
    )(emb0, emb1, factor1, count_w, value_w, io_w, pos_w)


_NW = 32
_TPW = BL // _NW
_CH = 128
_NCH = _TPW // _CH
_NCC = _NCH // 2

def _bf16_halves(w):
    v = plsc.bitcast(w, jnp.bfloat16)
    return plsc.unpack(v, format=plsc.PackFormat.INTERLEAVED)


def _sc_body(big_hbm, s_hbm, posw_hbm, ids_hbm, vci_hbm, pos_hbm, out_hbm,
             ids_loc, vci_loc, pos_loc,
             brow0, brow1, srow0, srow1, prow0, prow1, obuf0, obuf1,
             g00, g10, g20, g01, g11, g21, o0, o1):
    wid = lax.axis_index("s") * 2 + lax.axis_index("c")
    wbase = wid * _TPW

    pltpu.sync_copy(ids_hbm.at[wid], ids_loc)
    pltpu.sync_copy(vci_hbm.at[wid], vci_loc)
    pltpu.sync_copy(pos_hbm.at[wid], pos_loc)

    sets = ((brow0, srow0, prow0, obuf0, g00, g10, g20, o0),
            (brow1, srow1, prow1, obuf1, g01, g11, g21, o1))

    def fire_gathers(c, bset):
        brow, srow, prow = bset[0], bset[1], bset[2]
        pltpu.async_copy(big_hbm.at[ids_loc.at[c]], brow, bset[4])
        pltpu.async_copy(s_hbm.at[vci_loc.at[c]], srow, bset[5])
        pltpu.async_copy(posw_hbm.at[pos_loc.at[c]], prow, bset[6])

    def wait_gathers(c, bset):
        pltpu.make_async_copy(big_hbm.at[ids_loc.at[c]], bset[0], bset[4]).wait()
        pltpu.make_async_copy(s_hbm.at[vci_loc.at[c]], bset[1], bset[5]).wait()
        pltpu.make_async_copy(posw_hbm.at[pos_loc.at[c]], bset[2], bset[6]).wait()

    fire_gathers(0, sets[0])
    fire_gathers(1, sets[1])

    def loop_body(cc, carry):
        for b in (0, 1):
            brow, srow, prow, obuf, _, _, _, osem = sets[b]
            c = 2 * cc + b
            base = wbase + c * _CH
            out_slice = out_hbm.at[pl.ds(base, _CH)]
            wait_gathers(c, sets[b])

            @pl.when(cc > 0)
            def _drain():
                pltpu.make_async_copy(obuf, out_slice, osem).wait()

            @plsc.parallel_loop(0, _CH, unroll=4)
            def row_body(t):
                for j in range(4):
                    sl = pl.ds(j * 16, 16)
                    sh = pl.ds(HIDDEN // 2 + j * 16, 16)
                    slo, shi = _bf16_halves(srow[t, sl])
                    plo, phi = _bf16_halves(prow[t, sl])
                    obuf[t, sl] = brow[t, sl] + slo + plo
                    obuf[t, sh] = brow[t, sh] + shi + phi
            pltpu.async_copy(obuf, out_slice, osem)

            @pl.when(cc < _NCC - 1)
            def _prefetch():
                fire_gathers(c + 2, sets[b])
        return carry

    lax.fori_loop(0, _NCC, loop_body, 0)

    for b in (0, 1):
        obuf, osem = sets[b][3], sets[b][7]
        pltpu.make_async_copy(obuf, out_hbm.at[pl.ds(wbase, _CH)], osem).wait()


_sc_gather = functools.partial(
    pl.kernel,
    out_type=jax.ShapeDtypeStruct((BL, HIDDEN), jnp.float32),
    mesh=plsc.VectorSubcoreMesh(core_axis_name="c", subcore_axis_name="s"),
    compiler_params=pltpu.CompilerParams(needs_layout_passes=False,
                                         use_tc_tiling_on_sc=False),
    scratch_types=[
        pltpu.VMEM((_NCH, _CH), jnp.int32),
        pltpu.VMEM((_NCH, _CH), jnp.int32),
        pltpu.VMEM((_NCH, _CH), jnp.int32),
        pltpu.VMEM((_CH, HIDDEN), jnp.float32),
        pltpu.VMEM((_CH, HIDDEN), jnp.float32),
        pltpu.VMEM((_CH, HIDDEN // 2), jnp.int32),
        pltpu.VMEM((_CH, HIDDEN // 2), jnp.int32),
        pltpu.VMEM((_CH, HIDDEN // 2), jnp.int32),
        pltpu.VMEM((_CH, HIDDEN // 2), jnp.int32),
        pltpu.VMEM((_CH, HIDDEN), jnp.float32),
        pltpu.VMEM((_CH, HIDDEN), jnp.float32),
        pltpu.SemaphoreType.DMA,
        pltpu.SemaphoreType.DMA,
        pltpu.SemaphoreType.DMA,
        pltpu.SemaphoreType.DMA,
        pltpu.SemaphoreType.DMA,
        pltpu.SemaphoreType.DMA,
        pltpu.SemaphoreType.DMA,
        pltpu.SemaphoreType.DMA,
    ],
)(_sc_body)


def kernel(input_ids, counts, values, io_flags, positions,
           emb0, emb1, factor1, value_w, count_w, pos_w, io_w):
    big, s_tab, pos_tab = _build_tables(emb0, emb1, factor1,
                                        count_w, value_w, io_w, pos_w)
    shp = (_NW, _NCH, _CH)
    vci = counts * 45 + values * 3 + io_flags
    out = _sc_gather(big, s_tab, pos_tab,
                     input_ids.reshape(shp), vci.reshape(shp),
                     positions.reshape(shp))
    return out.reshape(B, L, HIDDEN)


# speedup vs baseline: 2.7169x; 1.0085x over previous
import functools

import jax
import jax.numpy as jnp
from jax import lax
from jax.experimental import pallas as pl
from jax.experimental.pallas import tpu as pltpu
from jax.experimental.pallas import tpu_sc as plsc

B, L, HIDDEN = 1024, 200, 128
VOCAB = 100000
NB0 = 50000
BL = B * L


_BLK = 10000
_NBLK0 = NB0 // _BLK


def _pack_bf16_words(x):
    lo = lax.bitcast_convert_type(x[:, :HIDDEN // 2], jnp.int32)
    hi = lax.bitcast_convert_type(x[:, HIDDEN // 2:], jnp.int32)
    return lax.shift_right_logical(lo, 16) | (hi & -65536)


def _table_body(emb0_ref, emb1_ref, factor1_ref, count_ref, value_ref,
                io_ref, posw_ref, big_out, s_out, pos_out):
    i = pl.program_id(0)

    @pl.when(i < _NBLK0)
    def _copy():
        big_out[...] = emb0_ref[...]

    @pl.when(i >= _NBLK0)
    def _proj():
        big_out[...] = jnp.dot(emb1_ref[...], factor1_ref[...],
                               preferred_element_type=jnp.float32)

    @pl.when(i == 0)
    def _small():
        s = (count_ref[...][:, None, None, :]
             + value_ref[...][None, :, None, :]
             + io_ref[...][None, None, :, :])
        s_out[...] = _pack_bf16_words(s.reshape(675, HIDDEN))
        pos_out[...] = _pack_bf16_words(posw_ref[...])


def _build_tables(emb0, emb1, factor1, count_w, value_w, io_w, pos_w):
    zero = lambda i: (0, 0)
    return pl.pallas_call(
        _table_body,
        grid=(2 * _NBLK0,),
        in_specs=[
            pl.BlockSpec((_BLK, HIDDEN), lambda i: (jnp.minimum(i, _NBLK0 - 1), 0)),
            pl.BlockSpec((_BLK, 32), lambda i: (jnp.maximum(i - _NBLK0, 0), 0)),
            pl.BlockSpec((32, HIDDEN), zero),
            pl.BlockSpec((15, HIDDEN), zero),
            pl.BlockSpec((15, HIDDEN), zero),
            pl.BlockSpec((3, HIDDEN), zero),
            pl.BlockSpec((L, HIDDEN), zero),
        ],
        out_specs=(pl.BlockSpec((_BLK, HIDDEN), lambda i: (i, 0)),
                   pl.BlockSpec((675, HIDDEN // 2), zero),
                   pl.BlockSpec((L, HIDDEN // 2), zero)),
        out_shape=(jax.ShapeDtypeStruct((VOCAB, HIDDEN), jnp.float32),
                   jax.ShapeDtypeStruct((675, HIDDEN // 2), jnp.int32),
                   jax.ShapeDtypeStruct((L, HIDDEN // 2), jnp.int32)),
    )(emb0, emb1, factor1, count_w, value_w, io_w, pos_w)


_NW = 32
_TPW = BL // _NW
_CH = 128
_NCH = _TPW // _CH
_NCC = _NCH // 2

def _bf16_halves(w):
    v = plsc.bitcast(w, jnp.bfloat16)
    return plsc.unpack(v, format=plsc.PackFormat.INTERLEAVED)


def _sc_body(big_hbm, s_hbm, posw_hbm, ids_hbm, vci_hbm, pos_hbm, out_hbm,
             ids_loc, vci_loc, pos_loc,
             brow0, brow1, srow0, srow1, prow0, prow1, obuf0, obuf1,
             g00, g10, g20, g01, g11, g21, o0, o1):
    wid = lax.axis_index("s") * 2 + lax.axis_index("c")
    wbase = wid * _TPW

    pltpu.sync_copy(ids_hbm.at[wid], ids_loc)
    pltpu.sync_copy(vci_hbm.at[wid], vci_loc)
    pltpu.sync_copy(pos_hbm.at[wid], pos_loc)

    sets = ((brow0, srow0, prow0, obuf0, g00, g10, g20, o0),
            (brow1, srow1, prow1, obuf1, g01, g11, g21, o1))

    def fire_gathers(c, bset):
        brow, srow, prow = bset[0], bset[1], bset[2]
        pltpu.async_copy(big_hbm.at[ids_loc.at[c]], brow, bset[4])
        pltpu.async_copy(s_hbm.at[vci_loc.at[c]], srow, bset[5])
        pltpu.async_copy(posw_hbm.at[pos_loc.at[c]], prow, bset[6])

    def wait_gathers(c, bset):
        pltpu.make_async_copy(big_hbm.at[ids_loc.at[c]], bset[0], bset[4]).wait()
        pltpu.make_async_copy(s_hbm.at[vci_loc.at[c]], bset[1], bset[5]).wait()
        pltpu.make_async_copy(posw_hbm.at[pos_loc.at[c]], bset[2], bset[6]).wait()

    fire_gathers(0, sets[0])
    fire_gathers(1, sets[1])

    def loop_body(cc, carry):
        for b in (0, 1):
            brow, srow, prow, obuf, _, _, _, osem = sets[b]
            c = 2 * cc + b
            base = wbase + c * _CH
            out_slice = out_hbm.at[pl.ds(base, _CH)]
            wait_gathers(c, sets[b])

            @pl.when(cc > 0)
            def _drain():
                pltpu.make_async_copy(obuf, out_slice, osem).wait()

            @plsc.parallel_loop(0, _CH, unroll=4)
            def row_body(t):
                for j in range(4):
                    sl = pl.ds(j * 16, 16)
                    sh = pl.ds(HIDDEN // 2 + j * 16, 16)
                    slo, shi = _bf16_halves(srow[t, sl])
                    plo, phi = _bf16_halves(prow[t, sl])
                    obuf[t, sl] = brow[t, sl] + slo + plo
                    obuf[t, sh] = brow[t, sh] + shi + phi
            pltpu.async_copy(obuf, out_slice, osem)

            @pl.when(cc < _NCC - 1)
            def _prefetch():
                fire_gathers(c + 2, sets[b])
        return carry

    lax.fori_loop(0, _NCC, loop_body, 0)

    for b in (0, 1):
        obuf, osem = sets[b][3], sets[b][7]
        pltpu.make_async_copy(obuf, out_hbm.at[pl.ds(wbase, _CH)], osem).wait()


_sc_gather = functools.partial(
    pl.kernel,
    out_type=jax.ShapeDtypeStruct((BL, HIDDEN), jnp.float32),
    mesh=plsc.VectorSubcoreMesh(core_axis_name="c", subcore_axis_name="s"),
    compiler_params=pltpu.CompilerParams(needs_layout_passes=False,
                                         use_tc_tiling_on_sc=False),
    scratch_types=[
        pltpu.VMEM((_NCH, _CH), jnp.int32),
        pltpu.VMEM((_NCH, _CH), jnp.int32),
        pltpu.VMEM((_NCH, _CH), jnp.int32),
        pltpu.VMEM((_CH, HIDDEN), jnp.float32),
        pltpu.VMEM((_CH, HIDDEN), jnp.float32),
        pltpu.VMEM((_CH, HIDDEN // 2), jnp.int32),
        pltpu.VMEM((_CH, HIDDEN // 2), jnp.int32),
        pltpu.VMEM((_CH, HIDDEN // 2), jnp.int32),
        pltpu.VMEM((_CH, HIDDEN // 2), jnp.int32),
        pltpu.VMEM((_CH, HIDDEN), jnp.float32),
        pltpu.VMEM((_CH, HIDDEN), jnp.float32),
        pltpu.SemaphoreType.DMA,
        pltpu.SemaphoreType.DMA,
        pltpu.SemaphoreType.DMA,
        pltpu.SemaphoreType.DMA,
        pltpu.SemaphoreType.DMA,
        pltpu.SemaphoreType.DMA,
        pltpu.SemaphoreType.DMA,
        pltpu.SemaphoreType.DMA,
    ],
)(_sc_body)


def kernel(input_ids, counts, values, io_flags, positions,
           emb0, emb1, factor1, value_w, count_w, pos_w, io_w):
    big, s_tab, pos_tab = _build_tables(emb0, emb1, factor1,
                                        count_w, value_w, io_w, pos_w)
    shp = (_NW, _NCH, _CH)
    vci = counts * 45 + values * 3 + io_flags
    out = _sc_gather(big, s_tab, pos_tab,
                     input_ids.reshape(shp), vci.reshape(shp),
                     positions.reshape(shp))
    return out.reshape(B, L, HIDDEN)
